# baseline (device time: 27112 ns/iter reference)
import os

import jax
import jax.numpy as jnp
from jax import lax
from jax.experimental import pallas as pl
from jax.experimental.pallas import tpu as pltpu

N_DEV = 16
N_GRP = int(os.environ.get("K_NGRP", "4"))
_CAST = os.environ.get("K_CAST", "fp8")

_DeviceIdType = getattr(pl, "DeviceIdType", None) or pltpu.DeviceIdType


def _mxu(v):
    if v.dtype == jnp.float32 and _CAST == "fp8":
        return v.astype(jnp.float8_e4m3fn)
    if v.dtype == jnp.float32 and _CAST == "bf16":
        return v.astype(jnp.bfloat16)
    return v


def _dot(a, b):
    return lax.dot_general(
        a, b, (((1,), (0,)), ((), ())),
        preferred_element_type=jnp.float32,
        precision=lax.Precision.DEFAULT,
    )


def kernel(x, w_mat, scale_x, scale_w):
    m_per, k = x.shape
    _, n = w_mat.shape
    n_per = n // N_DEV
    m_out = m_per * N_DEV
    d_per_g = N_DEV // N_GRP
    n_g = n // N_GRP

    def body(x_ref, w_ref, sx_ref, sw_ref, out_ref, wbuf_ref, stage_ref,
             rstage_ref, send_sems, recv_sems, load_sems):
        my = lax.axis_index("i")

        loads = []

        def start_load(g):
            cp = pltpu.make_async_copy(
                w_ref.at[:, g * n_g:(g + 1) * n_g],
                wbuf_ref.at[g % 2],
                load_sems.at[g % 2],
            )
            cp.start()
            loads.append(cp)

        start_load(0)

        barrier_sem = pltpu.get_barrier_semaphore()
        for kk in range(4):
            pl.semaphore_signal(barrier_sem, inc=1,
                                device_id=(lax.rem(my + (1 << kk), N_DEV),),
                                device_id_type=_DeviceIdType.MESH)
            pl.semaphore_wait(barrier_sem, 1)

        scale = sx_ref[0] * sw_ref[0]
        xv = _mxu(x_ref[...])

        def send_block(dst, blk, descs):
            @pl.when(dst == my)
            def _():
                out_ref[pl.ds(my * m_per, m_per), :] = blk

            @pl.when(dst != my)
            def _():
                stage_ref[dst] = blk.astype(jnp.bfloat16)

            rdma = pltpu.make_async_remote_copy(
                src_ref=stage_ref.at[dst],
                dst_ref=rstage_ref.at[my],
                send_sem=send_sems.at[dst],
                recv_sem=recv_sems.at[my],
                device_id=(dst,),
                device_id_type=_DeviceIdType.MESH,
            )

            @pl.when(dst != my)
            def _():
                rdma.start()

            descs.append((dst, rdma))

        descs = []
        for g in range(N_GRP):
            if g + 1 < N_GRP:
                start_load(g + 1)
            loads[g].wait()
            wv = _mxu(wbuf_ref[g % 2])
            acc = _dot(xv, wv) * scale
            for j in range(d_per_g):
                dst = g * d_per_g + j
                send_block(dst, acc[:, j * n_per:(j + 1) * n_per], descs)

        for src in range(N_DEV):
            recv = pltpu.make_async_remote_copy(
                src_ref=stage_ref.at[src],
                dst_ref=rstage_ref.at[src],
                send_sem=send_sems.at[src],
                recv_sem=recv_sems.at[src],
                device_id=(src,),
                device_id_type=_DeviceIdType.MESH,
            )

            @pl.when(src != my)
            def _():
                recv.wait_recv()
                out_ref[pl.ds(src * m_per, m_per), :] = (
                    rstage_ref[src].astype(jnp.float32))

        for dst, rdma in descs:
            @pl.when(dst != my)
            def _():
                rdma.wait_send()

    return pl.pallas_call(
        body,
        out_shape=jax.ShapeDtypeStruct((m_out, n_per), jnp.float32),
        in_specs=[
            pl.BlockSpec(memory_space=pltpu.VMEM),
            pl.BlockSpec(memory_space=pl.ANY),
            pl.BlockSpec(memory_space=pltpu.SMEM),
            pl.BlockSpec(memory_space=pltpu.SMEM),
        ],
        out_specs=pl.BlockSpec(memory_space=pltpu.VMEM),
        scratch_shapes=[
            pltpu.VMEM((2, k, n_g), jnp.float32),
            pltpu.VMEM((N_DEV, m_per, n_per), jnp.bfloat16),
            pltpu.VMEM((N_DEV, m_per, n_per), jnp.bfloat16),
            pltpu.SemaphoreType.DMA((N_DEV,)),
            pltpu.SemaphoreType.DMA((N_DEV,)),
            pltpu.SemaphoreType.DMA((2,)),
        ],
        compiler_params=pltpu.CompilerParams(
            vmem_limit_bytes=100 * 1024 * 1024,
            collective_id=0,
        ),
    )(x, w_mat, scale_x, scale_w)
